# Initial kernel scaffold; baseline (speedup 1.0000x reference)
import jax
import jax.numpy as jnp
from jax.experimental import pallas as pl


def kernel(hidden_state, embedding_weight):
    raise NotImplementedError("write your pallas kernel here")


# speedup vs baseline: 1.1247x; 1.1247x over previous
import jax
import jax.numpy as jnp
from jax.experimental import pallas as pl

_CODEBOOK = 8192
_DIM = 256
_TOK = 1024
_CHUNK = 1024
_NCHUNK = _CODEBOOK // _CHUNK
_GROUPS = ((0, 2736), (2736, 5472), (5472, _CODEBOOK))


def _vq_body(hs_ref, e_ref, hs2_ref, e2_ref, out_ref):
    hs = hs_ref[0]
    hs2 = hs2_ref[0]

    inf = jnp.full((1, _TOK), jnp.inf, jnp.float32)
    zero = jnp.zeros((1, _TOK), jnp.int32)
    best = [[inf, zero] for _ in _GROUPS]

    for c in range(_NCHUNK):
        e = e_ref[pl.ds(c * _CHUNK, _CHUNK), :]
        e2 = e2_ref[pl.ds(c * _CHUNK, _CHUNK), :]
        norms = hs2 + e2
        scores = norms - 2.0 * jnp.dot(
            e, hs, preferred_element_type=jnp.float32)
        rows = jax.lax.broadcasted_iota(jnp.int32, scores.shape, 0)
        base = c * _CHUNK
        for g, (glo, ghi) in enumerate(_GROUPS):
            lo, hi = max(glo - base, 0), min(ghi - base, _CHUNK)
            if hi <= lo:
                continue
            if lo > 0 or hi < _CHUNK:
                s = jnp.where((rows >= lo) & (rows < hi), scores, jnp.inf)
            else:
                s = scores
            cmin = jnp.min(s, axis=0, keepdims=True)
            cidx = jnp.min(
                jnp.where(s == cmin, rows + base, jnp.int32(_CODEBOOK)),
                axis=0, keepdims=True)
            upd = cmin < best[g][0]
            best[g][0] = jnp.where(upd, cmin, best[g][0])
            best[g][1] = jnp.where(upd, cidx, best[g][1])

    accv, acci = best[0]
    for g in range(1, len(_GROUPS)):
        mg, ag = best[g]
        accq = accv.astype(jnp.bfloat16).astype(jnp.float32)
        keep = (accq < mg) | ((accq == mg) & (acci < ag))
        accv = jnp.where(keep, accq, mg)
        acci = jnp.where(keep, acci, ag)
    out_ref[0] = acci


def kernel(hidden_state, embedding_weight):
    b, t, c, h, w = hidden_state.shape
    n_slabs = b * t
    tok = h * w

    hs2 = jnp.sum(
        jnp.transpose(hidden_state, (0, 1, 3, 4, 2)) ** 2, axis=-1
    ).reshape(n_slabs, 1, tok)
    e2 = jnp.sum(embedding_weight ** 2, axis=1).reshape(_CODEBOOK, 1)

    hs_bf = hidden_state.astype(jnp.bfloat16).reshape(n_slabs, c, tok)
    e_bf = embedding_weight.astype(jnp.bfloat16)

    out = pl.pallas_call(
        _vq_body,
        grid=(n_slabs,),
        in_specs=[
            pl.BlockSpec((1, c, tok), lambda i: (i, 0, 0)),
            pl.BlockSpec((_CODEBOOK, _DIM), lambda i: (0, 0)),
            pl.BlockSpec((1, 1, tok), lambda i: (i, 0, 0)),
            pl.BlockSpec((_CODEBOOK, 1), lambda i: (0, 0)),
        ],
        out_specs=pl.BlockSpec((1, 1, tok), lambda i: (i, 0, 0)),
        out_shape=jax.ShapeDtypeStruct((n_slabs, 1, tok), jnp.int32),
    )(hs_bf, e_bf, hs2, e2)

    return out.reshape(b, t, h, w)


# speedup vs baseline: 1.1578x; 1.0295x over previous
import jax
import jax.numpy as jnp
from jax.experimental import pallas as pl

_CODEBOOK = 8192
_DIM = 256
_TOK = 1024
_CHUNK = 512
_NCHUNK = _CODEBOOK // _CHUNK
_GROUPS = ((0, 2736), (2736, 5472), (5472, _CODEBOOK))


def _vq_body(hs_ref, e_ref, hs2_ref, e2_ref, out_ref):
    hs = hs_ref[0]
    hs2 = hs2_ref[0]

    inf = jnp.full((_CHUNK, _TOK), jnp.inf, jnp.float32)
    zero = jnp.zeros((_CHUNK, _TOK), jnp.int32)
    accv = [inf, inf, inf]
    accc = [zero, zero, zero]
    rows = jax.lax.broadcasted_iota(jnp.int32, (_CHUNK, _TOK), 0)

    for c in range(_NCHUNK):
        e = e_ref[pl.ds(c * _CHUNK, _CHUNK), :]
        e2 = e2_ref[pl.ds(c * _CHUNK, _CHUNK), :]
        norms = hs2 + e2
        scores = norms - 2.0 * jnp.dot(
            e, hs, preferred_element_type=jnp.float32)
        base = c * _CHUNK
        for g, (glo, ghi) in enumerate(_GROUPS):
            lo, hi = max(glo - base, 0), min(ghi - base, _CHUNK)
            if hi <= lo:
                continue
            if lo > 0 or hi < _CHUNK:
                s = jnp.where((rows >= lo) & (rows < hi), scores, jnp.inf)
            else:
                s = scores
            upd = s < accv[g]
            accv[g] = jnp.where(upd, s, accv[g])
            accc[g] = jnp.where(upd, jnp.int32(c), accc[g])

    best = []
    for g in range(len(_GROUPS)):
        gmin = jnp.min(accv[g], axis=0, keepdims=True)
        gidx = jnp.min(
            jnp.where(accv[g] == gmin, accc[g] * _CHUNK + rows,
                      jnp.int32(_CODEBOOK)),
            axis=0, keepdims=True)
        best.append((gmin, gidx))

    accv_f, acci_f = best[0]
    for g in range(1, len(_GROUPS)):
        mg, ag = best[g]
        accq = accv_f.astype(jnp.bfloat16).astype(jnp.float32)
        keep = (accq < mg) | ((accq == mg) & (acci_f < ag))
        accv_f = jnp.where(keep, accq, mg)
        acci_f = jnp.where(keep, acci_f, ag)
    out_ref[0] = acci_f


def kernel(hidden_state, embedding_weight):
    b, t, c, h, w = hidden_state.shape
    n_slabs = b * t
    tok = h * w

    hs2 = jnp.sum(
        jnp.transpose(hidden_state, (0, 1, 3, 4, 2)) ** 2, axis=-1
    ).reshape(n_slabs, 1, tok)
    e2 = jnp.sum(embedding_weight ** 2, axis=1).reshape(_CODEBOOK, 1)

    hs_bf = hidden_state.astype(jnp.bfloat16).reshape(n_slabs, c, tok)
    e_bf = embedding_weight.astype(jnp.bfloat16)

    out = pl.pallas_call(
        _vq_body,
        grid=(n_slabs,),
        in_specs=[
            pl.BlockSpec((1, c, tok), lambda i: (i, 0, 0)),
            pl.BlockSpec((_CODEBOOK, _DIM), lambda i: (0, 0)),
            pl.BlockSpec((1, 1, tok), lambda i: (i, 0, 0)),
            pl.BlockSpec((_CODEBOOK, 1), lambda i: (0, 0)),
        ],
        out_specs=pl.BlockSpec((1, 1, tok), lambda i: (i, 0, 0)),
        out_shape=jax.ShapeDtypeStruct((n_slabs, 1, tok), jnp.int32),
    )(hs_bf, e_bf, hs2, e2)

    return out.reshape(b, t, h, w)
